# baseline (speedup 1.0000x reference)
import functools

import jax
import jax.numpy as jnp
from jax import lax
from jax.experimental import pallas as pl
from jax.experimental.pallas import tpu as pltpu
from jax.experimental.pallas import tpu_sc as plsc

_N_USERS = 25000
_N_ITEMS = 25000
_N = _N_USERS + _N_ITEMS
_H = 32
_E = 800000
_C = 128
_NSUB = 16
_TCH = 408
_CHUNKS = _TCH * _NSUB
_EP = _CHUNKS * _C
_GC = 12
_NG = _TCH // _GC
_R = 6
_D = 4
_RPT = _N // _NSUB
_ZB = 125

_GDN = lax.GatherDimensionNumbers(
    offset_dims=(), collapsed_slice_dims=(0,), start_index_map=(0,))


def _scale_chunk(rows, vals_blk, j):
    for g in range(_C // 16):
        vals16 = vals_blk[j, pl.ds(g * 16, 16)]
        for r in range(16):
            v16 = lax.gather(vals16, jnp.full((16, 1), r, jnp.int32), _GDN,
                             (1,), mode=lax.GatherScatterMode.PROMISE_IN_BOUNDS)
            row = g * 16 + r
            rows[row, pl.ds(0, 16)] = rows[row, pl.ds(0, 16)] * v16
            rows[row, pl.ds(16, 16)] = rows[row, pl.ds(16, 16)] * v16


def _layer_body(src_hbm, dst_hbm, vals_hbm, elo_hbm, ehi_hbm,
                outlo_hbm, outhi_hbm,
                acc, src_blk, dst_blk, vals_blk, rows, gsem, ssem):
    cid = lax.axis_index("c")
    sid = lax.axis_index("s")

    zeros16 = jnp.zeros((16,), jnp.float32)

    def rb_zero(i, carry):
        rows[0][i, pl.ds(0, 16)] = zeros16
        rows[0][i, pl.ds(16, 16)] = zeros16
        return carry

    lax.fori_loop(0, _C, rb_zero, 0)

    def acc_zero(j, carry):
        pltpu.sync_copy(rows[0].at[pl.ds(0, _ZB)],
                        acc.at[pl.ds(sid * _RPT + j * _ZB, _ZB)])
        return carry

    lax.fori_loop(0, _RPT // _ZB, acc_zero, 0)
    plsc.subcore_barrier()

    def run_half(e_hbm, out_hbm):
        def group_body(g, carry):
            grow = sid * _TCH + g * _GC
            pltpu.sync_copy(src_hbm.at[pl.ds(grow, _GC)], src_blk)
            pltpu.sync_copy(dst_hbm.at[pl.ds(grow, _GC)], dst_blk)
            pltpu.sync_copy(vals_hbm.at[pl.ds(grow, _GC)], vals_blk)
            for b in range(_D):
                pltpu.async_copy(e_hbm.at[src_blk.at[b]], rows[b], gsem[b])

            def ring_body(q, c2):
                for b in range(_R):
                    j = q * _R + b
                    jn = j + _D
                    bn = (b + _D) % _R

                    @pl.when(jnp.logical_and(jn >= _R, jn < _GC))
                    def _(jn=jn, bn=bn):
                        pltpu.make_async_copy(
                            rows[bn], acc.at[dst_blk.at[0]], ssem[bn]).wait()
                        pltpu.async_copy(
                            e_hbm.at[src_blk.at[jn]], rows[bn], gsem[bn])

                    @pl.when(jn < _R)
                    def _(jn=jn, bn=bn):
                        pltpu.async_copy(
                            e_hbm.at[src_blk.at[jn]], rows[bn], gsem[bn])

                    pltpu.make_async_copy(
                        e_hbm.at[src_blk.at[j]], rows[b], gsem[b]).wait()
                    _scale_chunk(rows[b], vals_blk, j)
                    pltpu.async_copy(rows[b], acc.at[dst_blk.at[j]], ssem[b],
                                     add=True)
                return c2

            lax.fori_loop(0, _GC // _R, ring_body, 0)
            for b in range(_R):
                pltpu.make_async_copy(rows[b], acc.at[dst_blk.at[0]], ssem[b]).wait()
            return carry

        lax.fori_loop(0, _NG, group_body, 0)
        plsc.subcore_barrier()
        off = sid * _RPT
        pltpu.sync_copy(acc.at[pl.ds(off, _RPT)], out_hbm.at[pl.ds(off, _RPT)])

    @pl.when(cid == 0)
    def _():
        run_half(elo_hbm, outlo_hbm)

    @pl.when(cid == 1)
    def _():
        run_half(ehi_hbm, outhi_hbm)


@functools.cache
def _make_layer():
    mesh = plsc.VectorSubcoreMesh(core_axis_name="c", subcore_axis_name="s")
    return pl.kernel(
        _layer_body,
        out_type=[jax.ShapeDtypeStruct((_N, _H), jnp.float32)] * 2,
        mesh=mesh,
        scratch_types=[
            pltpu.VMEM_SHARED((_N, _H), jnp.float32),
            pltpu.VMEM((_GC, _C), jnp.int32),
            pltpu.VMEM((_GC, _C), jnp.int32),
            pltpu.VMEM((_GC, _C), jnp.float32),
            [pltpu.VMEM((_C, _H), jnp.float32)] * _R,
            [pltpu.SemaphoreType.DMA] * _R,
            [pltpu.SemaphoreType.DMA] * _R,
        ],
        compiler_params=pltpu.CompilerParams(use_tc_tiling_on_sc=False),
    )


def _mean_body(a0, a1, a2, a3, b0, b1, b2, b3, olo, ohi):
    olo[...] = (a0[...] + a1[...] + a2[...] + a3[...]) * 0.25
    ohi[...] = (b0[...] + b1[...] + b2[...] + b3[...]) * 0.25


_BLK = 400


@functools.cache
def _make_mean():
    spec = pl.BlockSpec((_BLK, _H), lambda i: (i, 0))
    return pl.pallas_call(
        _mean_body,
        grid=(_N // _BLK,),
        in_specs=[spec] * 8,
        out_specs=[spec] * 2,
        out_shape=[jax.ShapeDtypeStruct((_N, _H), jnp.float32)] * 2,
    )


def kernel(adj_indices, adj_values, user_emb, item_emb):
    pad = _EP - _E
    src = jnp.concatenate([adj_indices[1], jnp.zeros((pad,), jnp.int32)])
    dst = jnp.concatenate([adj_indices[0], jnp.zeros((pad,), jnp.int32)])
    vals = jnp.concatenate([adj_values, jnp.zeros((pad,), jnp.float32)])
    src2 = src.reshape(_CHUNKS, _C)
    dst2 = dst.reshape(_CHUNKS, _C)
    vals2 = vals.reshape(_CHUNKS, _C)
    e0lo = jnp.concatenate([user_emb[:, :_H], item_emb[:, :_H]], axis=0)
    e0hi = jnp.concatenate([user_emb[:, _H:], item_emb[:, _H:]], axis=0)
    layer = _make_layer()
    e1lo, e1hi = layer(src2, dst2, vals2, e0lo, e0hi)
    e2lo, e2hi = layer(src2, dst2, vals2, e1lo, e1hi)
    e3lo, e3hi = layer(src2, dst2, vals2, e2lo, e2hi)
    flo, fhi = _make_mean()(e0lo, e1lo, e2lo, e3lo, e0hi, e1hi, e2hi, e3hi)
    users = jnp.concatenate([flo[:_N_USERS], fhi[:_N_USERS]], axis=1)
    items = jnp.concatenate([flo[_N_USERS:], fhi[_N_USERS:]], axis=1)
    return users, items


# speedup vs baseline: 2.2791x; 2.2791x over previous
import functools

import jax
import jax.numpy as jnp
from jax import lax
from jax.experimental import pallas as pl
from jax.experimental.pallas import tpu as pltpu
from jax.experimental.pallas import tpu_sc as plsc

_N_USERS = 25000
_N_ITEMS = 25000
_N = _N_USERS + _N_ITEMS
_H = 32
_E = 800000
_C = 128
_NSUB = 16
_TCH = 392
_CHUNKS = _TCH * _NSUB
_EP = _CHUNKS * _C
_GC = 28
_NG = _TCH // _GC
_R = 4
_D = 2
_S = 2
_RPT = _N // _NSUB
_ZB = 125

_GDN = lax.GatherDimensionNumbers(
    offset_dims=(), collapsed_slice_dims=(0,), start_index_map=(0,))


def _scale_chunk(gbuf, sbuf, vals_blk, j):
    for g in range(_C // 16):
        vals16 = vals_blk[j, pl.ds(g * 16, 16)]
        for r in range(16):
            v16 = lax.gather(vals16, jnp.full((16, 1), r, jnp.int32), _GDN,
                             (1,), mode=lax.GatherScatterMode.PROMISE_IN_BOUNDS)
            row = g * 16 + r
            a, b = plsc.unpack(gbuf[row, :],
                               format=plsc.PackFormat.INTERLEAVED)
            sbuf[row, pl.ds(0, 16)] = a * v16
            sbuf[row, pl.ds(16, 16)] = b * v16


def _layer_body(src_hbm, dst_hbm, vals_hbm, tlo_hbm, thi_hbm,
                flo_hbm, blo_hbm, fhi_hbm, bhi_hbm,
                acc, src_blk, dst_blk, vals_blk, gbufs, sbufs, gsem, ssem):
    cid = lax.axis_index("c")
    sid = lax.axis_index("s")

    zeros16 = jnp.zeros((16,), jnp.float32)

    def sb_zero(i, carry):
        sbufs[0][i, pl.ds(0, 16)] = zeros16
        sbufs[0][i, pl.ds(16, 16)] = zeros16
        return carry

    lax.fori_loop(0, _C, sb_zero, 0)

    def acc_zero(j, carry):
        pltpu.sync_copy(sbufs[0].at[pl.ds(0, _ZB)],
                        acc.at[pl.ds(sid * _RPT + j * _ZB, _ZB)])
        return carry

    lax.fori_loop(0, _RPT // _ZB, acc_zero, 0)
    plsc.subcore_barrier()

    def run_half(t_hbm, f_hbm, b_hbm):
        def group_body(g, carry):
            grow = sid * _TCH + g * _GC
            pltpu.sync_copy(src_hbm.at[pl.ds(grow, _GC)], src_blk)
            pltpu.sync_copy(dst_hbm.at[pl.ds(grow, _GC)], dst_blk)
            pltpu.sync_copy(vals_hbm.at[pl.ds(grow, _GC)], vals_blk)
            for b in range(_D):
                pltpu.async_copy(t_hbm.at[src_blk.at[b]], gbufs[b], gsem[b])

            def ring_body(q, c2):
                for b in range(_R):
                    j = q * _R + b
                    jn = j + _D
                    bn = (b + _D) % _R
                    c = b % _S

                    @pl.when(jn < _GC)
                    def _(jn=jn, bn=bn):
                        pltpu.async_copy(
                            t_hbm.at[src_blk.at[jn]], gbufs[bn], gsem[bn])

                    pltpu.make_async_copy(
                        t_hbm.at[src_blk.at[j]], gbufs[b], gsem[b]).wait()

                    @pl.when(j >= _S)
                    def _(c=c):
                        pltpu.make_async_copy(
                            sbufs[c], acc.at[dst_blk.at[0]], ssem[c]).wait()

                    _scale_chunk(gbufs[b], sbufs[c], vals_blk, j)
                    pltpu.async_copy(sbufs[c], acc.at[dst_blk.at[j]], ssem[c],
                                     add=True)
                return c2

            lax.fori_loop(0, _GC // _R, ring_body, 0)
            for c in range(_S):
                pltpu.make_async_copy(
                    sbufs[c], acc.at[dst_blk.at[0]], ssem[c]).wait()
            return carry

        lax.fori_loop(0, _NG, group_body, 0)
        plsc.subcore_barrier()

        off = sid * _RPT
        pltpu.sync_copy(acc.at[pl.ds(off, _RPT)], f_hbm.at[pl.ds(off, _RPT)])

        def wb(jj, carry):
            o2 = off + jj * _ZB
            pltpu.sync_copy(acc.at[pl.ds(o2, _ZB)], sbufs[0].at[pl.ds(0, _ZB)])

            def pack_row(r, c3):
                a = sbufs[0][r, pl.ds(0, 16)]
                b = sbufs[0][r, pl.ds(16, 16)]
                gbufs[0][r, :] = plsc.pack(
                    a, b, format=plsc.PackFormat.INTERLEAVED)
                return c3

            lax.fori_loop(0, _ZB, pack_row, 0)
            pltpu.sync_copy(gbufs[0].at[pl.ds(0, _ZB)],
                            b_hbm.at[pl.ds(o2, _ZB)])
            return carry

        lax.fori_loop(0, _RPT // _ZB, wb, 0)

    @pl.when(cid == 0)
    def _():
        run_half(tlo_hbm, flo_hbm, blo_hbm)

    @pl.when(cid == 1)
    def _():
        run_half(thi_hbm, fhi_hbm, bhi_hbm)


@functools.cache
def _make_layer():
    mesh = plsc.VectorSubcoreMesh(core_axis_name="c", subcore_axis_name="s")
    return pl.kernel(
        _layer_body,
        out_type=[
            jax.ShapeDtypeStruct((_N, _H), jnp.float32),
            jax.ShapeDtypeStruct((_N, _H), jnp.bfloat16),
            jax.ShapeDtypeStruct((_N, _H), jnp.float32),
            jax.ShapeDtypeStruct((_N, _H), jnp.bfloat16),
        ],
        mesh=mesh,
        scratch_types=[
            pltpu.VMEM_SHARED((_N, _H), jnp.float32),
            pltpu.VMEM((_GC, _C), jnp.int32),
            pltpu.VMEM((_GC, _C), jnp.int32),
            pltpu.VMEM((_GC, _C), jnp.float32),
            [pltpu.VMEM((_C, _H), jnp.bfloat16)] * _R,
            [pltpu.VMEM((_C, _H), jnp.float32)] * _S,
            [pltpu.SemaphoreType.DMA] * _R,
            [pltpu.SemaphoreType.DMA] * _S,
        ],
        compiler_params=pltpu.CompilerParams(use_tc_tiling_on_sc=False,
                                             needs_layout_passes=False),
    )


def _mean_body(a0, a1, a2, a3, b0, b1, b2, b3, olo, ohi):
    olo[...] = (a0[...] + a1[...] + a2[...] + a3[...]) * 0.25
    ohi[...] = (b0[...] + b1[...] + b2[...] + b3[...]) * 0.25


_BLK = 400


@functools.cache
def _make_mean():
    spec = pl.BlockSpec((_BLK, _H), lambda i: (i, 0))
    return pl.pallas_call(
        _mean_body,
        grid=(_N // _BLK,),
        in_specs=[spec] * 8,
        out_specs=[spec] * 2,
        out_shape=[jax.ShapeDtypeStruct((_N, _H), jnp.float32)] * 2,
    )


def _permute(x):
    return jnp.concatenate([x[:, 0::2], x[:, 1::2]], axis=1)


def _unpermute(x):
    return x.reshape(-1, 2, 16).transpose(0, 2, 1).reshape(-1, 2 * 16)


def kernel(adj_indices, adj_values, user_emb, item_emb):
    pad = _EP - _E
    src = jnp.concatenate([adj_indices[1], jnp.zeros((pad,), jnp.int32)])
    dst = jnp.concatenate([adj_indices[0], jnp.zeros((pad,), jnp.int32)])
    vals = jnp.concatenate([adj_values, jnp.zeros((pad,), jnp.float32)])
    src2 = src.reshape(_CHUNKS, _C)
    dst2 = dst.reshape(_CHUNKS, _C)
    vals2 = vals.reshape(_CHUNKS, _C)
    e0lo = jnp.concatenate([user_emb[:, :_H], item_emb[:, :_H]], axis=0)
    e0hi = jnp.concatenate([user_emb[:, _H:], item_emb[:, _H:]], axis=0)
    t0lo = e0lo.astype(jnp.bfloat16)
    t0hi = e0hi.astype(jnp.bfloat16)
    layer = _make_layer()
    f1lo, t1lo, f1hi, t1hi = layer(src2, dst2, vals2, t0lo, t0hi)
    f2lo, t2lo, f2hi, t2hi = layer(src2, dst2, vals2, t1lo, t1hi)
    f3lo, _, f3hi, _ = layer(src2, dst2, vals2, t2lo, t2hi)
    flo, fhi = _make_mean()(_permute(e0lo), f1lo, f2lo, f3lo,
                            _permute(e0hi), f1hi, f2hi, f3hi)
    flo = _unpermute(flo)
    fhi = _unpermute(fhi)
    users = jnp.concatenate([flo[:_N_USERS], fhi[:_N_USERS]], axis=1)
    items = jnp.concatenate([flo[_N_USERS:], fhi[_N_USERS:]], axis=1)
    return users, items


# speedup vs baseline: 2.4176x; 1.0608x over previous
import functools

import jax
import jax.numpy as jnp
from jax import lax
from jax.experimental import pallas as pl
from jax.experimental.pallas import tpu as pltpu
from jax.experimental.pallas import tpu_sc as plsc

_N_USERS = 25000
_N_ITEMS = 25000
_N = _N_USERS + _N_ITEMS
_H = 32
_E = 800000
_C = 128
_NSUB = 16
_TCH = 392
_CHUNKS = _TCH * _NSUB
_EP = _CHUNKS * _C
_GC = 28
_NG = _TCH // _GC
_R = 4
_D = 3
_S = 2
_RPT = _N // _NSUB
_ZB = 125

_GDN = lax.GatherDimensionNumbers(
    offset_dims=(), collapsed_slice_dims=(0,), start_index_map=(0,))


def _scale_chunk(gbuf, sbuf, vals_blk, j):
    for g in range(_C // 16):
        vals16 = vals_blk[j, pl.ds(g * 16, 16)]
        for r in range(16):
            v16 = lax.gather(vals16, jnp.full((16, 1), r, jnp.int32), _GDN,
                             (1,), mode=lax.GatherScatterMode.PROMISE_IN_BOUNDS)
            row = g * 16 + r
            a, b = plsc.unpack(gbuf[row, :],
                               format=plsc.PackFormat.INTERLEAVED)
            sbuf[row, pl.ds(0, 16)] = a * v16
            sbuf[row, pl.ds(16, 16)] = b * v16


def _layer_body(src_hbm, dst_hbm, vals_hbm, tlo_hbm, thi_hbm,
                flo_hbm, blo_hbm, fhi_hbm, bhi_hbm,
                acc, src_blk, dst_blk, vals_blk, gbufs, sbufs, gsem, ssem):
    cid = lax.axis_index("c")
    sid = lax.axis_index("s")

    zeros16 = jnp.zeros((16,), jnp.float32)

    def sb_zero(i, carry):
        sbufs[0][i, pl.ds(0, 16)] = zeros16
        sbufs[0][i, pl.ds(16, 16)] = zeros16
        return carry

    lax.fori_loop(0, _C, sb_zero, 0)

    def acc_zero(j, carry):
        pltpu.sync_copy(sbufs[0].at[pl.ds(0, _ZB)],
                        acc.at[pl.ds(sid * _RPT + j * _ZB, _ZB)])
        return carry

    lax.fori_loop(0, _RPT // _ZB, acc_zero, 0)
    plsc.subcore_barrier()

    def run_half(t_hbm, f_hbm, b_hbm):
        def group_body(g, carry):
            grow = sid * _TCH + g * _GC
            pltpu.sync_copy(src_hbm.at[pl.ds(grow, _GC)], src_blk)
            pltpu.sync_copy(dst_hbm.at[pl.ds(grow, _GC)], dst_blk)
            pltpu.sync_copy(vals_hbm.at[pl.ds(grow, _GC)], vals_blk)
            for b in range(_D):
                pltpu.async_copy(t_hbm.at[src_blk.at[b]], gbufs[b], gsem[b])

            def ring_body(q, c2):
                for b in range(_R):
                    j = q * _R + b
                    jn = j + _D
                    bn = (b + _D) % _R
                    c = b % _S

                    @pl.when(jn < _GC)
                    def _(jn=jn, bn=bn):
                        pltpu.async_copy(
                            t_hbm.at[src_blk.at[jn]], gbufs[bn], gsem[bn])

                    pltpu.make_async_copy(
                        t_hbm.at[src_blk.at[j]], gbufs[b], gsem[b]).wait()

                    @pl.when(j >= _S)
                    def _(c=c):
                        pltpu.make_async_copy(
                            sbufs[c], acc.at[dst_blk.at[0]], ssem[c]).wait()

                    _scale_chunk(gbufs[b], sbufs[c], vals_blk, j)
                    pltpu.async_copy(sbufs[c], acc.at[dst_blk.at[j]], ssem[c],
                                     add=True)
                return c2

            lax.fori_loop(0, _GC // _R, ring_body, 0)
            for c in range(_S):
                pltpu.make_async_copy(
                    sbufs[c], acc.at[dst_blk.at[0]], ssem[c]).wait()
            return carry

        lax.fori_loop(0, _NG, group_body, 0)
        plsc.subcore_barrier()

        off = sid * _RPT
        pltpu.sync_copy(acc.at[pl.ds(off, _RPT)], f_hbm.at[pl.ds(off, _RPT)])

        def wb(jj, carry):
            o2 = off + jj * _ZB
            pltpu.sync_copy(acc.at[pl.ds(o2, _ZB)], sbufs[0].at[pl.ds(0, _ZB)])

            def pack_row(r, c3):
                a = sbufs[0][r, pl.ds(0, 16)]
                b = sbufs[0][r, pl.ds(16, 16)]
                gbufs[0][r, :] = plsc.pack(
                    a, b, format=plsc.PackFormat.INTERLEAVED)
                return c3

            lax.fori_loop(0, _ZB, pack_row, 0)
            pltpu.sync_copy(gbufs[0].at[pl.ds(0, _ZB)],
                            b_hbm.at[pl.ds(o2, _ZB)])
            return carry

        lax.fori_loop(0, _RPT // _ZB, wb, 0)

    @pl.when(cid == 0)
    def _():
        run_half(tlo_hbm, flo_hbm, blo_hbm)

    @pl.when(cid == 1)
    def _():
        run_half(thi_hbm, fhi_hbm, bhi_hbm)


@functools.cache
def _make_layer():
    mesh = plsc.VectorSubcoreMesh(core_axis_name="c", subcore_axis_name="s")
    return pl.kernel(
        _layer_body,
        out_type=[
            jax.ShapeDtypeStruct((_N, _H), jnp.float32),
            jax.ShapeDtypeStruct((_N, _H), jnp.bfloat16),
            jax.ShapeDtypeStruct((_N, _H), jnp.float32),
            jax.ShapeDtypeStruct((_N, _H), jnp.bfloat16),
        ],
        mesh=mesh,
        scratch_types=[
            pltpu.VMEM_SHARED((_N, _H), jnp.float32),
            pltpu.VMEM((_GC, _C), jnp.int32),
            pltpu.VMEM((_GC, _C), jnp.int32),
            pltpu.VMEM((_GC, _C), jnp.float32),
            [pltpu.VMEM((_C, _H), jnp.bfloat16)] * _R,
            [pltpu.VMEM((_C, _H), jnp.float32)] * _S,
            [pltpu.SemaphoreType.DMA] * _R,
            [pltpu.SemaphoreType.DMA] * _S,
        ],
        compiler_params=pltpu.CompilerParams(use_tc_tiling_on_sc=False,
                                             needs_layout_passes=False),
    )


def _mean_body(a0, a1, a2, a3, b0, b1, b2, b3, olo, ohi):
    olo[...] = (a0[...] + a1[...] + a2[...] + a3[...]) * 0.25
    ohi[...] = (b0[...] + b1[...] + b2[...] + b3[...]) * 0.25


_BLK = 400


@functools.cache
def _make_mean():
    spec = pl.BlockSpec((_BLK, _H), lambda i: (i, 0))
    return pl.pallas_call(
        _mean_body,
        grid=(_N // _BLK,),
        in_specs=[spec] * 8,
        out_specs=[spec] * 2,
        out_shape=[jax.ShapeDtypeStruct((_N, _H), jnp.float32)] * 2,
    )


def _permute(x):
    return jnp.concatenate([x[:, 0::2], x[:, 1::2]], axis=1)


def _unpermute(x):
    return x.reshape(-1, 2, 16).transpose(0, 2, 1).reshape(-1, 2 * 16)


def kernel(adj_indices, adj_values, user_emb, item_emb):
    pad = _EP - _E
    src = jnp.concatenate([adj_indices[1], jnp.zeros((pad,), jnp.int32)])
    dst = jnp.concatenate([adj_indices[0], jnp.zeros((pad,), jnp.int32)])
    vals = jnp.concatenate([adj_values, jnp.zeros((pad,), jnp.float32)])
    src2 = src.reshape(_CHUNKS, _C)
    dst2 = dst.reshape(_CHUNKS, _C)
    vals2 = vals.reshape(_CHUNKS, _C)
    e0lo = jnp.concatenate([user_emb[:, :_H], item_emb[:, :_H]], axis=0)
    e0hi = jnp.concatenate([user_emb[:, _H:], item_emb[:, _H:]], axis=0)
    t0lo = e0lo.astype(jnp.bfloat16)
    t0hi = e0hi.astype(jnp.bfloat16)
    layer = _make_layer()
    f1lo, t1lo, f1hi, t1hi = layer(src2, dst2, vals2, t0lo, t0hi)
    f2lo, t2lo, f2hi, t2hi = layer(src2, dst2, vals2, t1lo, t1hi)
    f3lo, _, f3hi, _ = layer(src2, dst2, vals2, t2lo, t2hi)
    flo, fhi = _make_mean()(_permute(e0lo), f1lo, f2lo, f3lo,
                            _permute(e0hi), f1hi, f2hi, f3hi)
    flo = _unpermute(flo)
    fhi = _unpermute(fhi)
    users = jnp.concatenate([flo[:_N_USERS], fhi[:_N_USERS]], axis=1)
    items = jnp.concatenate([flo[_N_USERS:], fhi[_N_USERS:]], axis=1)
    return users, items


# speedup vs baseline: 2.4298x; 1.0050x over previous
import functools

import jax
import jax.numpy as jnp
from jax import lax
from jax.experimental import pallas as pl
from jax.experimental.pallas import tpu as pltpu
from jax.experimental.pallas import tpu_sc as plsc

_N_USERS = 25000
_N_ITEMS = 25000
_N = _N_USERS + _N_ITEMS
_H = 32
_E = 800000
_C = 128
_NSUB = 16
_TCH = 392
_CHUNKS = _TCH * _NSUB
_EP = _CHUNKS * _C
_GC = 28
_NG = _TCH // _GC
_R = 4
_D = 2
_RPT = _N // _NSUB
_ZB = 125

_GDN = lax.GatherDimensionNumbers(
    offset_dims=(), collapsed_slice_dims=(0,), start_index_map=(0,))


def _scale_chunk(rows, vals_blk, j):
    for g in range(_C // 16):
        vals16 = vals_blk[j, pl.ds(g * 16, 16)]
        for r in range(16):
            v16 = lax.gather(vals16, jnp.full((16, 1), r, jnp.int32), _GDN,
                             (1,), mode=lax.GatherScatterMode.PROMISE_IN_BOUNDS)
            row = g * 16 + r
            rows[row, pl.ds(0, 16)] = rows[row, pl.ds(0, 16)] * v16
            rows[row, pl.ds(16, 16)] = rows[row, pl.ds(16, 16)] * v16


def _layer_body(src_hbm, dst_hbm, vals_hbm, elo_hbm, ehi_hbm,
                outlo_hbm, outhi_hbm,
                acc, src_blk, dst_blk, vals_blk, rows, gsem, ssem):
    cid = lax.axis_index("c")
    sid = lax.axis_index("s")

    zeros16 = jnp.zeros((16,), jnp.float32)

    def rb_zero(i, carry):
        rows[0][i, pl.ds(0, 16)] = zeros16
        rows[0][i, pl.ds(16, 16)] = zeros16
        return carry

    lax.fori_loop(0, _C, rb_zero, 0)

    def acc_zero(j, carry):
        pltpu.sync_copy(rows[0].at[pl.ds(0, _ZB)],
                        acc.at[pl.ds(sid * _RPT + j * _ZB, _ZB)])
        return carry

    lax.fori_loop(0, _RPT // _ZB, acc_zero, 0)
    plsc.subcore_barrier()

    def run_half(e_hbm, out_hbm):
        def group_body(g, carry):
            grow = sid * _TCH + g * _GC
            pltpu.sync_copy(src_hbm.at[pl.ds(grow, _GC)], src_blk)
            pltpu.sync_copy(dst_hbm.at[pl.ds(grow, _GC)], dst_blk)
            pltpu.sync_copy(vals_hbm.at[pl.ds(grow, _GC)], vals_blk)
            for b in range(_D):
                pltpu.async_copy(e_hbm.at[src_blk.at[b]], rows[b], gsem[b])

            def ring_body(q, c2):
                for b in range(_R):
                    j = q * _R + b
                    jn = j + _D
                    bn = (b + _D) % _R

                    @pl.when(jnp.logical_and(jn >= _R, jn < _GC))
                    def _(jn=jn, bn=bn):
                        pltpu.make_async_copy(
                            rows[bn], acc.at[dst_blk.at[0]], ssem[bn]).wait()
                        pltpu.async_copy(
                            e_hbm.at[src_blk.at[jn]], rows[bn], gsem[bn])

                    @pl.when(jn < _R)
                    def _(jn=jn, bn=bn):
                        pltpu.async_copy(
                            e_hbm.at[src_blk.at[jn]], rows[bn], gsem[bn])

                    pltpu.make_async_copy(
                        e_hbm.at[src_blk.at[j]], rows[b], gsem[b]).wait()
                    _scale_chunk(rows[b], vals_blk, j)
                    pltpu.async_copy(rows[b], acc.at[dst_blk.at[j]], ssem[b],
                                     add=True)
                return c2

            lax.fori_loop(0, _GC // _R, ring_body, 0)
            for b in range(_R):
                pltpu.make_async_copy(rows[b], acc.at[dst_blk.at[0]], ssem[b]).wait()
            return carry

        lax.fori_loop(0, _NG, group_body, 0)
        plsc.subcore_barrier()
        off = sid * _RPT
        pltpu.sync_copy(acc.at[pl.ds(off, _RPT)], out_hbm.at[pl.ds(off, _RPT)])

    @pl.when(cid == 0)
    def _():
        run_half(elo_hbm, outlo_hbm)

    @pl.when(cid == 1)
    def _():
        run_half(ehi_hbm, outhi_hbm)


@functools.cache
def _make_layer():
    mesh = plsc.VectorSubcoreMesh(core_axis_name="c", subcore_axis_name="s")
    return pl.kernel(
        _layer_body,
        out_type=[jax.ShapeDtypeStruct((_N, _H), jnp.float32)] * 2,
        mesh=mesh,
        scratch_types=[
            pltpu.VMEM_SHARED((_N, _H), jnp.float32),
            pltpu.VMEM((_GC, _C), jnp.int32),
            pltpu.VMEM((_GC, _C), jnp.int32),
            pltpu.VMEM((_GC, _C), jnp.float32),
            [pltpu.VMEM((_C, _H), jnp.float32)] * _R,
            [pltpu.SemaphoreType.DMA] * _R,
            [pltpu.SemaphoreType.DMA] * _R,
        ],
        compiler_params=pltpu.CompilerParams(use_tc_tiling_on_sc=False),
    )


def _mean_body(a0, a1, a2, a3, b0, b1, b2, b3, olo, ohi):
    olo[...] = (a0[...] + a1[...] + a2[...] + a3[...]) * 0.25
    ohi[...] = (b0[...] + b1[...] + b2[...] + b3[...]) * 0.25


_BLK = 400


@functools.cache
def _make_mean():
    spec = pl.BlockSpec((_BLK, _H), lambda i: (i, 0))
    return pl.pallas_call(
        _mean_body,
        grid=(_N // _BLK,),
        in_specs=[spec] * 8,
        out_specs=[spec] * 2,
        out_shape=[jax.ShapeDtypeStruct((_N, _H), jnp.float32)] * 2,
    )


def kernel(adj_indices, adj_values, user_emb, item_emb):
    pad = _EP - _E
    src = jnp.concatenate([adj_indices[1], jnp.zeros((pad,), jnp.int32)])
    dst = jnp.concatenate([adj_indices[0], jnp.zeros((pad,), jnp.int32)])
    vals = jnp.concatenate([adj_values, jnp.zeros((pad,), jnp.float32)])
    src2 = src.reshape(_CHUNKS, _C)
    dst2 = dst.reshape(_CHUNKS, _C)
    vals2 = vals.reshape(_CHUNKS, _C)
    e0lo = jnp.concatenate([user_emb[:, :_H], item_emb[:, :_H]], axis=0)
    e0hi = jnp.concatenate([user_emb[:, _H:], item_emb[:, _H:]], axis=0)
    layer = _make_layer()
    e1lo, e1hi = layer(src2, dst2, vals2, e0lo, e0hi)
    e2lo, e2hi = layer(src2, dst2, vals2, e1lo, e1hi)
    e3lo, e3hi = layer(src2, dst2, vals2, e2lo, e2hi)
    flo, fhi = _make_mean()(e0lo, e1lo, e2lo, e3lo, e0hi, e1hi, e2hi, e3hi)
    users = jnp.concatenate([flo[:_N_USERS], fhi[:_N_USERS]], axis=1)
    items = jnp.concatenate([flo[_N_USERS:], fhi[_N_USERS:]], axis=1)
    return users, items
